# baseline (speedup 1.0000x reference)
import functools

import jax
import jax.numpy as jnp
from jax import lax
from jax.experimental import pallas as pl
from jax.experimental.pallas import tpu as pltpu
from jax.experimental.pallas import tpu_sc as plsc

B = 16384
E = 16
NC = 2
NS = 16
NW = NC * NS
BPW = B // NW
CH = 128
NCH = BPW // CH
GRP = BPW // E


def _gmf_body(user_hbm, item_hbm, ut_hbm, it_hbm, out_hbm,
              uidx_v, iidx_v, urows_v, irows_v, out_v, sem):
    wid = lax.axis_index("s") * NC + lax.axis_index("c")
    base = wid * BPW

    for c in range(NCH):
        pltpu.sync_copy(user_hbm.at[pl.ds(base + c * CH, CH)], uidx_v.at[c])
        pltpu.sync_copy(item_hbm.at[pl.ds(base + c * CH, CH)], iidx_v.at[c])

    copies = []
    for c in range(NCH):
        copies.append(pltpu.async_copy(ut_hbm.at[uidx_v.at[c]], urows_v.at[c], sem))
        copies.append(pltpu.async_copy(it_hbm.at[iidx_v.at[c]], irows_v.at[c], sem))
    for cp in copies:
        cp.wait()

    lane = lax.iota(jnp.int32, 16)

    def group(g, carry):
        c = g // (CH // E)
        r0 = (g % (CH // E)) * E
        ridx = r0 + lane
        cvec = jnp.full((16,), 0, jnp.int32) + c
        acc = jnp.zeros((16,), jnp.float32)
        for d in range(E):
            dvec = jnp.full((16,), d, jnp.int32)
            u = plsc.load_gather(urows_v, [cvec, ridx, dvec])
            v = plsc.load_gather(irows_v, [cvec, ridx, dvec])
            acc = acc + u * v
        out_v[pl.ds(g * E, E)] = acc
        return carry

    lax.fori_loop(0, GRP, group, 0)

    pltpu.sync_copy(out_v, out_hbm.at[pl.ds(base, BPW)])


_gmf = functools.partial(
    pl.kernel,
    mesh=plsc.VectorSubcoreMesh(core_axis_name="c", subcore_axis_name="s"),
    out_type=jax.ShapeDtypeStruct((B,), jnp.float32),
    scratch_types=[
        pltpu.VMEM((NCH, CH), jnp.int32),
        pltpu.VMEM((NCH, CH), jnp.int32),
        pltpu.VMEM((NCH, CH, E), jnp.float32),
        pltpu.VMEM((NCH, CH, E), jnp.float32),
        pltpu.VMEM((BPW,), jnp.float32),
        pltpu.SemaphoreType.DMA,
    ],
    compiler_params=pltpu.CompilerParams(
        needs_layout_passes=False, use_tc_tiling_on_sc=False
    ),
)(_gmf_body)


def kernel(user, item, embed_user_GMF, embed_item_GMF):
    user = user.astype(jnp.int32)
    item = item.astype(jnp.int32)
    return _gmf(user, item, embed_user_GMF, embed_item_GMF)
